# Initial kernel scaffold; baseline (speedup 1.0000x reference)
import jax
import jax.numpy as jnp
from jax.experimental import pallas as pl


def kernel(indices, table):
    raise NotImplementedError("write your pallas kernel here")


# speedup vs baseline: 1.5056x; 1.5056x over previous
import functools

import jax
import jax.numpy as jnp
from jax import lax
from jax.experimental import pallas as pl
from jax.experimental.pallas import tpu as pltpu
from jax.experimental.pallas import tpu_sc as plsc

EMBED_DIM = 32
CHUNK = 2048


def _gather_kernel(n_total, n_chunks_per_worker, num_cores):
    mesh = plsc.VectorSubcoreMesh(core_axis_name="c", subcore_axis_name="s")
    b_per_w = n_chunks_per_worker * CHUNK

    @functools.partial(
        pl.kernel,
        mesh=mesh,
        out_type=jax.ShapeDtypeStruct((n_total, EMBED_DIM), jnp.float32),
        scratch_types=[
            pltpu.VMEM((CHUNK,), jnp.int32),
            pltpu.VMEM((CHUNK, EMBED_DIM), jnp.float32),
            pltpu.SemaphoreType.DMA,
        ],
        compiler_params=pltpu.CompilerParams(use_tc_tiling_on_sc=False),
    )
    def k(idx_hbm, table_hbm, out_hbm, idx_v, rows_v, sem):
        wid = lax.axis_index("s") * num_cores + lax.axis_index("c")
        base = wid * b_per_w

        def body(i, carry):
            off = base + i * CHUNK
            pltpu.sync_copy(idx_hbm.at[pl.ds(off, CHUNK)], idx_v)
            pltpu.async_copy(table_hbm.at[idx_v], rows_v, sem).wait()
            pltpu.sync_copy(rows_v, out_hbm.at[pl.ds(off, CHUNK)])
            return carry

        lax.fori_loop(0, n_chunks_per_worker, body, 0)

    return k


def kernel(indices, table):
    batch, hist = indices.shape
    n_total = batch * hist
    info = plsc.get_sparse_core_info()
    num_workers = info.num_cores * info.num_subcores
    assert n_total % (num_workers * CHUNK) == 0
    n_chunks_per_worker = n_total // (num_workers * CHUNK)

    flat_idx = indices.reshape(n_total).astype(jnp.int32)
    out = _gather_kernel(n_total, n_chunks_per_worker, info.num_cores)(
        flat_idx, table
    )
    return out.reshape(batch, hist, EMBED_DIM)


# speedup vs baseline: 1.5135x; 1.0052x over previous
import functools

import jax
import jax.numpy as jnp
from jax import lax
from jax.experimental import pallas as pl
from jax.experimental.pallas import tpu as pltpu
from jax.experimental.pallas import tpu_sc as plsc

EMBED_DIM = 32
CHUNK = 1024
NBUF = 2


def _gather_kernel(n_total, n_chunks, num_cores):
    mesh = plsc.VectorSubcoreMesh(core_axis_name="c", subcore_axis_name="s")
    b_per_w = n_chunks * CHUNK

    @functools.partial(
        pl.kernel,
        mesh=mesh,
        out_type=jax.ShapeDtypeStruct((n_total, EMBED_DIM), jnp.float32),
        scratch_types=[
            pltpu.VMEM((b_per_w,), jnp.int32),
            pltpu.VMEM((CHUNK, EMBED_DIM), jnp.float32),
            pltpu.VMEM((CHUNK, EMBED_DIM), jnp.float32),
            pltpu.SemaphoreType.DMA,
            pltpu.SemaphoreType.DMA,
            pltpu.SemaphoreType.DMA,
            pltpu.SemaphoreType.DMA,
        ],
        compiler_params=pltpu.CompilerParams(use_tc_tiling_on_sc=False),
    )
    def k(idx_hbm, table_hbm, out_hbm, idx_v, rows0, rows1, g0, g1, w0, w1):
        wid = lax.axis_index("s") * num_cores + lax.axis_index("c")
        base = wid * b_per_w
        rows = (rows0, rows1)
        gsem = (g0, g1)
        wsem = (w0, w1)

        pltpu.sync_copy(idx_hbm.at[pl.ds(base, b_per_w)], idx_v)

        def start_gather(g, b):
            pltpu.async_copy(
                table_hbm.at[idx_v.at[pl.ds(g * CHUNK, CHUNK)]],
                rows[b],
                gsem[b],
            )

        def wait_gather(b):
            pltpu.make_async_copy(
                table_hbm.at[idx_v.at[pl.ds(0, CHUNK)]], rows[b], gsem[b]
            ).wait()

        def start_wb(g, b):
            pltpu.async_copy(
                rows[b], out_hbm.at[pl.ds(base + g * CHUNK, CHUNK)], wsem[b]
            )

        def wait_wb(b):
            pltpu.make_async_copy(
                rows[b], out_hbm.at[pl.ds(0, CHUNK)], wsem[b]
            ).wait()

        for b in range(NBUF):
            start_gather(b, b)

        def body(o, carry):
            for b in range(NBUF):
                g = o * NBUF + b
                wait_gather(b)
                start_wb(g, b)
                wait_wb(b)
                start_gather(g + NBUF, b)
            return carry

        lax.fori_loop(0, n_chunks // NBUF - 1, body, 0)

        for b in range(NBUF):
            wait_gather(b)
            start_wb(n_chunks - NBUF + b, b)
        for b in range(NBUF):
            wait_wb(b)

    return k


def kernel(indices, table):
    batch, hist = indices.shape
    n_total = batch * hist
    info = plsc.get_sparse_core_info()
    num_workers = info.num_cores * info.num_subcores
    assert n_total % (num_workers * CHUNK) == 0
    n_chunks = n_total // (num_workers * CHUNK)
    assert n_chunks % NBUF == 0 and n_chunks >= 2 * NBUF

    flat_idx = indices.reshape(n_total).astype(jnp.int32)
    out = _gather_kernel(n_total, n_chunks, info.num_cores)(flat_idx, table)
    return out.reshape(batch, hist, EMBED_DIM)
